# baseline (speedup 1.0000x reference)
import functools
import jax
import jax.numpy as jnp
from jax import lax
from jax.experimental import pallas as pl
from jax.experimental.pallas import tpu as pltpu
from jax.experimental.pallas import tpu_sc as plsc

_B = 1024
_M = 8
_K = 32
_D = 128
_OUT = 128
_TR = 512
_R = _B * _M
_SC_ROWS = 512
_TC_ROWS = _R - _SC_ROWS

_NW = 32
_RPW = _SC_ROWS // _NW
_CH = 4
_NCH = _RPW // _CH
_ROW_ELEMS = 2 * _K * _D


def _main_body(sv_ref, nb0_ref, nb1_ref, mk_ref, w_ref, b_ref, out_ref):
    x0 = nb0_ref[:, 0]
    x1 = nb1_ref[:, 0]
    m = mk_ref[...]
    w = w_ref[...]
    sv = sv_ref[...]

    scale = 1.0 / _K
    bdn = (((2,), (1,)), ((0,), (0,)))
    e0 = jax.lax.dot_general(m[:, None, :_K], x0, bdn,
                             preferred_element_type=jnp.float32)[:, 0] * scale
    e1 = jax.lax.dot_general(m[:, None, _K:], x1, bdn,
                             preferred_element_type=jnp.float32)[:, 0] * scale

    dn = (((1,), (1,)), ((), ()))
    acc = jax.lax.dot_general(sv, w[:, :_D], dn,
                              preferred_element_type=jnp.float32)
    acc += jax.lax.dot_general(e0, w[:, _D:2 * _D], dn,
                               preferred_element_type=jnp.float32)
    acc += jax.lax.dot_general(e1, w[:, 2 * _D:], dn,
                               preferred_element_type=jnp.float32)
    out_ref[...] = acc + b_ref[...]


def _tail_body(sv_ref, e_ref, w_ref, b_ref, out_ref):
    e = e_ref[...]
    w = w_ref[...]
    dn = (((1,), (1,)), ((), ()))
    acc = jax.lax.dot_general(sv_ref[...], w[:, :_D], dn,
                              preferred_element_type=jnp.float32)
    acc += jax.lax.dot_general(e[:, :_D], w[:, _D:2 * _D], dn,
                               preferred_element_type=jnp.float32)
    acc += jax.lax.dot_general(e[:, _D:], w[:, 2 * _D:], dn,
                               preferred_element_type=jnp.float32)
    out_ref[...] = acc + b_ref[...]


def _sc_mean(nb_hbm, mk_hbm, e_hbm, buf0, buf1, mbuf, est, sem0, sem1, msem):
    wid = lax.axis_index("s") * 2 + lax.axis_index("c")
    gbase = _TC_ROWS + wid * _RPW
    pltpu.async_copy(mk_hbm.at[pl.ds(gbase * 2 * _K, _RPW * 2 * _K)],
                     mbuf, msem).wait()
    bufs = (buf0, buf1)
    sems = (sem0, sem1)
    handles = [None, None]
    handles[0] = pltpu.async_copy(
        nb_hbm.at[pl.ds(gbase * _ROW_ELEMS, _CH * _ROW_ELEMS)], bufs[0],
        sems[0])
    for ch in range(_NCH):
        cur = ch % 2
        if ch + 1 < _NCH:
            nxt = (ch + 1) % 2
            handles[nxt] = pltpu.async_copy(
                nb_hbm.at[pl.ds((gbase + (ch + 1) * _CH) * _ROW_ELEMS,
                                _CH * _ROW_ELEMS)], bufs[nxt], sems[nxt])
        handles[cur].wait()
        buf = bufs[cur]
        for c in range(_CH):
            lrow = ch * _CH + c
            zero8 = (jnp.zeros((16,), jnp.float32),) * 8

            def group_sum(jlo, accs, c=c, lrow=lrow, buf=buf):
                idx0 = jnp.full((16,), lrow * 2 * _K + jlo, jnp.int32)

                def body(j, carry):
                    idx, accs = carry
                    msp = plsc.load_gather(mbuf, [idx])
                    off = (c * 2 * _K + jlo + j) * _D
                    new = tuple(
                        accs[v] + msp * buf[pl.ds(off + 16 * v, 16)]
                        for v in range(8))
                    return (idx + jnp.ones((16,), jnp.int32), new)

                return lax.fori_loop(0, _K, body, (idx0, accs))[1]

            acc0 = group_sum(0, zero8)
            acc1 = group_sum(_K, zero8)
            for v in range(8):
                est[c, pl.ds(16 * v, 16)] = acc0[v] * (1.0 / _K)
                est[c, pl.ds(_D + 16 * v, 16)] = acc1[v] * (1.0 / _K)
        pltpu.sync_copy(
            est, e_hbm.at[pl.ds(wid * _RPW + ch * _CH, _CH)])


def kernel(self_vectors, neighbor_vectors, masks, W, b):
    nb4 = neighbor_vectors.reshape(_R, 2, _K, _D)
    nb1 = neighbor_vectors.reshape(_R * _ROW_ELEMS)
    mk = masks.reshape(_R, 2 * _K)
    mk1 = masks.reshape(_R * 2 * _K)
    sv = self_vectors.reshape(_R, _D)
    b2 = b.reshape(1, _OUT)

    out_main = pl.pallas_call(
        _main_body,
        grid=(_TC_ROWS // _TR,),
        in_specs=[
            pl.BlockSpec((_TR, _D), lambda i: (i, 0)),
            pl.BlockSpec((_TR, 1, _K, _D), lambda i: (i, 0, 0, 0)),
            pl.BlockSpec((_TR, 1, _K, _D), lambda i: (i, 1, 0, 0)),
            pl.BlockSpec((_TR, 2 * _K), lambda i: (i, 0)),
            pl.BlockSpec((_OUT, 3 * _D), lambda i: (0, 0)),
            pl.BlockSpec((1, _OUT), lambda i: (0, 0)),
        ],
        out_specs=pl.BlockSpec((_TR, _OUT), lambda i: (i, 0)),
        out_shape=jax.ShapeDtypeStruct((_TC_ROWS, _OUT), jnp.float32),
    )(sv, nb4, nb4, mk, W, b2)

    mesh = plsc.VectorSubcoreMesh(core_axis_name="c", subcore_axis_name="s")
    sc_mean = functools.partial(
        pl.kernel, mesh=mesh,
        out_type=jax.ShapeDtypeStruct((_SC_ROWS, 2 * _D), jnp.float32),
        scratch_types=[
            pltpu.VMEM((_CH * _ROW_ELEMS,), jnp.float32),
            pltpu.VMEM((_CH * _ROW_ELEMS,), jnp.float32),
            pltpu.VMEM((_RPW * 2 * _K,), jnp.float32),
            pltpu.VMEM((_CH, 2 * _D), jnp.float32),
            pltpu.SemaphoreType.DMA,
            pltpu.SemaphoreType.DMA,
            pltpu.SemaphoreType.DMA,
        ],
        compiler_params=pltpu.CompilerParams(needs_layout_passes=False),
    )(_sc_mean)
    e_tail = sc_mean(nb1, mk1)

    base = _TC_ROWS // _TR
    out_tail = pl.pallas_call(
        _tail_body,
        grid=(_SC_ROWS // _TR,),
        in_specs=[
            pl.BlockSpec((_TR, _D), lambda i: (i + base, 0)),
            pl.BlockSpec((_TR, 2 * _D), lambda i: (i, 0)),
            pl.BlockSpec((_OUT, 3 * _D), lambda i: (0, 0)),
            pl.BlockSpec((1, _OUT), lambda i: (0, 0)),
        ],
        out_specs=pl.BlockSpec((_TR, _OUT), lambda i: (i, 0)),
        out_shape=jax.ShapeDtypeStruct((_SC_ROWS, _OUT), jnp.float32),
    )(sv, e_tail, W, b2)

    out = jnp.concatenate([out_main, out_tail], axis=0)
    return out.reshape(_B, _M, _OUT)


# speedup vs baseline: 1.3006x; 1.3006x over previous
import jax
import jax.numpy as jnp
from jax.experimental import pallas as pl

_B = 1024
_M = 8
_K = 32
_D = 128
_OUT = 128
_TR = 512


def _body(sv_ref, nb0_ref, nb1_ref, mk_ref, w_ref, b_ref, out_ref):
    x0 = nb0_ref[:, 0]
    x1 = nb1_ref[:, 0]
    m = mk_ref[...]
    w = w_ref[...]
    sv = sv_ref[...]

    scale = 1.0 / _K
    bdn = (((2,), (1,)), ((0,), (0,)))
    e0 = jax.lax.dot_general(m[:, None, :_K], x0, bdn,
                             preferred_element_type=jnp.float32)[:, 0] * scale
    e1 = jax.lax.dot_general(m[:, None, _K:], x1, bdn,
                             preferred_element_type=jnp.float32)[:, 0] * scale

    dn = (((1,), (1,)), ((), ()))
    acc = jax.lax.dot_general(sv, w[:, :_D], dn,
                              preferred_element_type=jnp.float32)
    acc += jax.lax.dot_general(e0, w[:, _D:2 * _D], dn,
                               preferred_element_type=jnp.float32)
    acc += jax.lax.dot_general(e1, w[:, 2 * _D:], dn,
                               preferred_element_type=jnp.float32)
    out_ref[...] = acc + b_ref[...]


def kernel(self_vectors, neighbor_vectors, masks, W, b):
    R = _B * _M
    nb = neighbor_vectors.reshape(R, 2, _K, _D)
    mk = masks.reshape(R, 2 * _K)
    sv = self_vectors.reshape(R, _D)
    b2 = b.reshape(1, _OUT)

    grid = (R // _TR,)
    out = pl.pallas_call(
        _body,
        grid=grid,
        in_specs=[
            pl.BlockSpec((_TR, _D), lambda i: (i, 0)),
            pl.BlockSpec((_TR, 1, _K, _D), lambda i: (i, 0, 0, 0)),
            pl.BlockSpec((_TR, 1, _K, _D), lambda i: (i, 1, 0, 0)),
            pl.BlockSpec((_TR, 2 * _K), lambda i: (i, 0)),
            pl.BlockSpec((_OUT, 3 * _D), lambda i: (0, 0)),
            pl.BlockSpec((1, _OUT), lambda i: (0, 0)),
        ],
        out_specs=pl.BlockSpec((_TR, _OUT), lambda i: (i, 0)),
        out_shape=jax.ShapeDtypeStruct((R, _OUT), jnp.float32),
    )(sv, nb, nb, mk, W, b2)
    return out.reshape(_B, _M, _OUT)
